# baseline (speedup 1.0000x reference)
import functools

import jax
import jax.numpy as jnp
from jax import lax
from jax.experimental import pallas as pl
from jax.experimental.pallas import tpu as pltpu
from jax.experimental.pallas import tpu_sc as plsc

E = 8
TOP_K = 2
D = 1024
FF = 2048
S = 2048
T = 2048
H = 12
BETA = 0.5
KL = int(0.1 * T)
KH = int(0.1 * T)

C = T
BT = 256
NBPE = C // BT
MAXB = E * NBPE
BS_A = 1024


def _attn_sum_body(a_ref, o_ref):
    h = pl.program_id(1)
    s = jnp.sum(a_ref[0, 0], axis=1, keepdims=True)
    o_ref[...] = jnp.where(h == 0, s, o_ref[...] + s)


def _attn_row_sums(attn):
    return pl.pallas_call(
        _attn_sum_body,
        grid=(S // BS_A, H),
        in_specs=[pl.BlockSpec((1, 1, BS_A, S), lambda i, h: (0, h, i, 0))],
        out_specs=pl.BlockSpec((BS_A, 1), lambda i, h: (i, 0)),
        out_shape=jax.ShapeDtypeStruct((S, 1), jnp.float32),
    )(attn)


def _router_body(x_ref, g_ref, logits_ref, rw0_ref, rw1_ref, s0_ref, s1_ref,
                 l1_ref):
    x = x_ref[...]
    g = g_ref[...]
    logits = lax.dot_general(x, g, (((1,), (1,)), ((), ())),
                             preferred_element_type=jnp.float32)
    logits_ref[...] = logits
    m = jnp.max(logits, axis=1, keepdims=True)
    ex = jnp.exp(logits - m)
    probs = ex / jnp.sum(ex, axis=1, keepdims=True)
    cols = lax.broadcasted_iota(jnp.int32, probs.shape, 1)
    p0 = jnp.max(probs, axis=1, keepdims=True)
    i0 = jnp.min(jnp.where(probs == p0, cols, E), axis=1, keepdims=True)
    probs2 = jnp.where(cols == i0, -1.0, probs)
    p1 = jnp.max(probs2, axis=1, keepdims=True)
    i1 = jnp.min(jnp.where(probs2 == p1, cols, E), axis=1, keepdims=True)
    rw0_ref[...] = p0
    rw1_ref[...] = p1
    s0_ref[...] = i0
    s1_ref[...] = i1
    l1_ref[...] = jnp.sum(jnp.abs(x), axis=1, keepdims=True)


def _router(x2d, gate_w):
    f1 = jax.ShapeDtypeStruct((T, 1), jnp.float32)
    i1 = jax.ShapeDtypeStruct((T, 1), jnp.int32)
    col = pl.BlockSpec((BT, 1), lambda i: (i, 0))
    return pl.pallas_call(
        _router_body,
        grid=(T // BT,),
        in_specs=[pl.BlockSpec((BT, D), lambda i: (i, 0)),
                  pl.BlockSpec((E, D), lambda i: (0, 0))],
        out_specs=[pl.BlockSpec((BT, E), lambda i: (i, 0)),
                   col, col, col, col, col],
        out_shape=[jax.ShapeDtypeStruct((T, E), jnp.float32),
                   f1, f1, i1, i1, f1],
    )(x2d, gate_w)


def _rank_body(rs_row_ref, rs_col_ref, l1_row_ref, l1_col_ref, rw0_ref,
               rw1_ref, w0_ref, w1_ref):
    i = pl.program_id(0)
    j_full = lax.broadcasted_iota(jnp.int32, (1, T), 1)
    i_col = lax.broadcasted_iota(jnp.int32, (BT, 1), 0) + i * BT
    real_row = (S - j_full).astype(jnp.float32)
    real_col = (S - i_col).astype(jnp.float32)
    s_full = rs_row_ref[...] / 12.0 / real_row * l1_row_ref[...]
    s_col = rs_col_ref[...] / 12.0 / real_col * l1_col_ref[...]
    gt = jnp.sum((s_full > s_col).astype(jnp.int32), axis=1, keepdims=True)
    lt = jnp.sum((s_full < s_col).astype(jnp.int32), axis=1, keepdims=True)
    eqb = jnp.sum(((s_full == s_col) & (j_full < i_col)).astype(jnp.int32),
                  axis=1, keepdims=True)
    keep = (gt + eqb) < KH
    drop = (lt + eqb) < KL
    rw0 = rw0_ref[...]
    rw1 = rw1_ref[...]
    new0 = jnp.ones_like(rw0)
    new1 = jnp.where(rw1 < BETA * rw0, 0.0, 1.0)
    new0 = jnp.where(drop, 0.0, new0)
    new1 = jnp.where(drop, 0.0, new1)
    new0 = jnp.where(keep, 1.0, new0)
    new1 = jnp.where(keep, 1.0, new1)
    rm0 = rw0 * new0
    rm1 = rw1 * new1
    denom = rm0 + rm1
    safe = jnp.where(denom > 0, denom, 1.0)
    w0_ref[...] = jnp.where(denom > 0, rm0 / safe, 0.0)
    w1_ref[...] = jnp.where(denom > 0, rm1 / safe, 0.0)


def _rank_weights(rs_row, rs_col, l1_row, l1_col, rw0, rw1):
    f1 = jax.ShapeDtypeStruct((T, 1), jnp.float32)
    col = pl.BlockSpec((BT, 1), lambda i: (i, 0))
    row = pl.BlockSpec((1, T), lambda i: (0, 0))
    return pl.pallas_call(
        _rank_body,
        grid=(T // BT,),
        in_specs=[row, col, row, col, col, col],
        out_specs=[col, col],
        out_shape=[f1, f1],
    )(rs_row, rs_col, l1_row, l1_col, rw0, rw1)


@functools.lru_cache(maxsize=None)
def _sc_mesh():
    return plsc.VectorSubcoreMesh(core_axis_name="c", subcore_axis_name="s")


def _dispatch_body(s0_h, s1_h, w0_h, w1_h, disp_h, pst_h, cnt_h, ws_h,
                   s0_v, s1_v, w0_v, w1_v, seg_v, p0_v, p1_v, cv_v, wsor_v):
    wid = lax.axis_index("c") * 16 + lax.axis_index("s")

    @pl.when(wid < E)
    def _():
        e = wid
        pltpu.sync_copy(s0_h, s0_v)
        pltpu.sync_copy(s1_h, s1_v)
        pltpu.sync_copy(w0_h, w0_v)
        pltpu.sync_copy(w1_h, w1_v)
        z = jnp.zeros((16,), jnp.int32)
        zf = jnp.zeros((16,), jnp.float32)

        def zbody(i, carry):
            seg_v[pl.ds(i * 16, 16)] = z
            p0_v[pl.ds(i * 16, 16)] = z
            p1_v[pl.ds(i * 16, 16)] = z
            wsor_v[pl.ds(i * 16, 16)] = zf
            return carry

        lax.fori_loop(0, T // 16, zbody, 0)

        def body(ci, cnt):
            lanes = ci * 16 + lax.iota(jnp.int32, 16)
            s0 = s0_v[pl.ds(ci * 16, 16)]
            w0 = w0_v[pl.ds(ci * 16, 16)]
            m0 = (s0 == e) & (w0 > 0.0)
            pos0 = cnt + plsc.cumsum(m0.astype(jnp.int32)) - 1
            plsc.store_scatter(seg_v, [pos0], lanes, mask=m0)
            plsc.store_scatter(wsor_v, [pos0], w0, mask=m0)
            plsc.store_scatter(p0_v, [lanes], pos0 + (e * C + 1), mask=m0)
            cnt = cnt + jnp.sum(m0.astype(jnp.int32))
            s1 = s1_v[pl.ds(ci * 16, 16)]
            w1 = w1_v[pl.ds(ci * 16, 16)]
            m1 = (s1 == e) & (w1 > 0.0)
            pos1 = cnt + plsc.cumsum(m1.astype(jnp.int32)) - 1
            plsc.store_scatter(seg_v, [pos1], lanes, mask=m1)
            plsc.store_scatter(wsor_v, [pos1], w1, mask=m1)
            plsc.store_scatter(p1_v, [lanes], pos1 + (e * C + 1), mask=m1)
            cnt = cnt + jnp.sum(m1.astype(jnp.int32))
            return cnt

        cnt = lax.fori_loop(0, T // 16, body, 0)
        cv_v[...] = jnp.full((16,), cnt, jnp.int32)
        pltpu.sync_copy(cv_v, cnt_h.at[e])
        pltpu.sync_copy(seg_v, disp_h.at[pl.ds(e * C, C)])
        pltpu.sync_copy(wsor_v, ws_h.at[pl.ds(e * C, C)])
        pltpu.sync_copy(p0_v, pst_h.at[e, 0])
        pltpu.sync_copy(p1_v, pst_h.at[e, 1])


@functools.lru_cache(maxsize=None)
def _dispatch_kernel():
    return pl.kernel(
        _dispatch_body,
        out_type=(jax.ShapeDtypeStruct((E * C,), jnp.int32),
                  jax.ShapeDtypeStruct((E, 2, T), jnp.int32),
                  jax.ShapeDtypeStruct((E, 16), jnp.int32),
                  jax.ShapeDtypeStruct((E * C + BT,), jnp.float32)),
        mesh=_sc_mesh(),
        compiler_params=pltpu.CompilerParams(needs_layout_passes=False),
        scratch_types=[pltpu.VMEM((T,), jnp.int32),
                       pltpu.VMEM((T,), jnp.int32),
                       pltpu.VMEM((T,), jnp.float32),
                       pltpu.VMEM((T,), jnp.float32),
                       pltpu.VMEM((C,), jnp.int32),
                       pltpu.VMEM((T,), jnp.int32),
                       pltpu.VMEM((T,), jnp.int32),
                       pltpu.VMEM((16,), jnp.int32),
                       pltpu.VMEM((C,), jnp.float32)],
    )


_GCH = 64


def _gather_body(x_h, disp_h, cnt_h, xs_h, idx_v, rows_v, cv_v, sem):
    wid = lax.axis_index("c") * 16 + lax.axis_index("s")
    e = wid // 4
    q = wid % 4
    pltpu.sync_copy(cnt_h.at[e], cv_v)
    cnt = cv_v[...][0]
    nslots = ((cnt + BT - 1) // BT) * BT

    def body(jj, carry):
        start = (jj * 4 + q) * _GCH

        @pl.when(start < nslots)
        def _():
            pltpu.sync_copy(disp_h.at[pl.ds(e * C + start, _GCH)], idx_v)
            pltpu.async_copy(x_h.at[idx_v], rows_v, sem).wait()
            pltpu.sync_copy(rows_v, xs_h.at[pl.ds(e * C + start, _GCH)])
        return carry

    lax.fori_loop(0, (C // 4) // _GCH, body, 0)


@functools.lru_cache(maxsize=None)
def _gather_x_kernel():
    return pl.kernel(
        _gather_body,
        out_type=jax.ShapeDtypeStruct((E * C + BT, D), jnp.float32),
        mesh=_sc_mesh(),
        compiler_params=pltpu.CompilerParams(needs_layout_passes=False),
        scratch_types=[pltpu.VMEM((_GCH,), jnp.int32),
                       pltpu.VMEM((_GCH, D), jnp.float32),
                       pltpu.VMEM((16,), jnp.int32),
                       pltpu.SemaphoreType.DMA],
    )


def _mlp_body(act_s, blk_s, wsel_s, x_ref, ws_ref, w1_ref, w3_ref, w2_ref,
              y_ref):
    g = pl.program_id(0)

    @pl.when(act_s[g] == 1)
    def _():
        x = x_ref[...].astype(jnp.bfloat16)
        w1b = w1_ref[0].astype(jnp.bfloat16)
        w3b = w3_ref[0].astype(jnp.bfloat16)
        w2b = w2_ref[0].astype(jnp.bfloat16)
        h1 = lax.dot_general(x, w1b, (((1,), (1,)), ((), ())),
                             preferred_element_type=jnp.float32)
        h3 = lax.dot_general(x, w3b, (((1,), (1,)), ((), ())),
                             preferred_element_type=jnp.float32)
        a = ((h1 * jax.nn.sigmoid(h1)) * h3).astype(jnp.bfloat16)
        y = lax.dot_general(a, w2b, (((1,), (1,)), ((), ())),
                            preferred_element_type=jnp.float32)
        y_ref[...] = y * ws_ref[...]

    @pl.when(act_s[g] == 2)
    def _():
        y_ref[...] = jnp.zeros((BT, D), jnp.float32)


def _grouped_mlp(act, blkid, wsel, xs, ws2d, w1, w3, w2):
    grid_spec = pltpu.PrefetchScalarGridSpec(
        num_scalar_prefetch=3,
        grid=(MAXB + 1,),
        in_specs=[
            pl.BlockSpec((BT, D), lambda g, a, b, w: (b[g], 0)),
            pl.BlockSpec((BT, 1), lambda g, a, b, w: (b[g], 0)),
            pl.BlockSpec((1, FF, D), lambda g, a, b, w: (w[g], 0, 0)),
            pl.BlockSpec((1, FF, D), lambda g, a, b, w: (w[g], 0, 0)),
            pl.BlockSpec((1, D, FF), lambda g, a, b, w: (w[g], 0, 0)),
        ],
        out_specs=pl.BlockSpec((BT, D), lambda g, a, b, w: (b[g], 0)),
    )
    return pl.pallas_call(
        _mlp_body,
        grid_spec=grid_spec,
        out_shape=jax.ShapeDtypeStruct(((MAXB + 1) * BT, D), jnp.float32),
        compiler_params=pltpu.CompilerParams(vmem_limit_bytes=100 * 1024 * 1024),
    )(act, blkid, wsel, xs, ws2d, w1, w3, w2)


def _blockmap_body(cnt_ref, act_ref, blk_ref, wsel_ref):
    c = cnt_ref[...][:, 0:1]
    nb = (c + BT - 1) // BT
    er = lax.broadcasted_iota(jnp.int32, (E, E), 0)
    ec = lax.broadcasted_iota(jnp.int32, (E, E), 1)
    nbr = jnp.broadcast_to(nb.reshape(1, E), (E, E))
    cum_incl = jnp.sum(jnp.where(ec <= er, nbr, 0), axis=1, keepdims=True)
    g = lax.broadcasted_iota(jnp.int32, (1, 128), 1)
    ge = g >= cum_incl
    eg = jnp.sum(ge.astype(jnp.int32), axis=0, keepdims=True)
    cume = jnp.sum(jnp.where(ge, jnp.broadcast_to(nb, (E, 128)), 0),
                   axis=0, keepdims=True)
    tot = jnp.sum(nb)
    egc = jnp.minimum(eg, E - 1)
    act = jnp.where(g < tot, 1, 0)
    act = jnp.where(g == MAXB, 2, act)
    act_ref[...] = act
    blk_ref[...] = jnp.where(act == 1, egc * NBPE + (g - cume), MAXB)
    wsel_ref[...] = jnp.where(act == 1, egc, 0)


def _blockmap(counts16):
    o = jax.ShapeDtypeStruct((1, 128), jnp.int32)
    full = pl.BlockSpec((1, 128), lambda: (0, 0))
    return pl.pallas_call(
        _blockmap_body,
        in_specs=[pl.BlockSpec((E, 16), lambda: (0, 0))],
        out_specs=[full, full, full],
        out_shape=[o, o, o],
    )(counts16)


def _pred_body(pst_ref, p2_ref):
    s = jnp.sum(pst_ref[...], axis=0) - 1
    p2_ref[...] = jnp.where(s < 0, E * C, s)


def _reduce_pstage(pstage):
    return pl.pallas_call(
        _pred_body,
        in_specs=[pl.BlockSpec((E, 2, T), lambda: (0, 0, 0))],
        out_specs=pl.BlockSpec((2, T), lambda: (0, 0)),
        out_shape=jax.ShapeDtypeStruct((2, T), jnp.int32),
    )(pstage)


_CCH = 32


def _combine_body(y_h, p2_h, out_h,
                  p0_v, p1_v, idx_v, rows_v, acc_v, sem0, sem1):
    wid = lax.axis_index("c") * 16 + lax.axis_index("s")
    t0 = wid * 64
    pltpu.sync_copy(p2_h.at[0, pl.ds(t0, 64)], p0_v)
    pltpu.sync_copy(p2_h.at[1, pl.ds(t0, 64)], p1_v)

    for half in range(2):
        tbase = t0 + half * _CCH
        for cchunk in range(_CCH // 16):
            lanes = (half * _CCH + cchunk * 16) + lax.iota(jnp.int32, 16)
            dpos = 2 * (cchunk * 16 + lax.iota(jnp.int32, 16))
            v0 = plsc.load_gather(p0_v, [lanes])
            v1 = plsc.load_gather(p1_v, [lanes])
            plsc.store_scatter(idx_v, [dpos], v0)
            plsc.store_scatter(idx_v, [dpos + 1], v1)

        pltpu.async_copy(y_h.at[idx_v], rows_v, sem0).wait()

        def accum(r, carry):
            for i in range(D // 16):
                sl = pl.ds(i * 16, 16)
                acc_v[r, sl] = rows_v[2 * r, sl] + rows_v[2 * r + 1, sl]
            return carry

        lax.fori_loop(0, _CCH, accum, 0)
        pltpu.sync_copy(acc_v, out_h.at[0, pl.ds(tbase, _CCH)])


@functools.lru_cache(maxsize=None)
def _combine_kernel():
    return pl.kernel(
        _combine_body,
        out_type=jax.ShapeDtypeStruct((1, S, D), jnp.float32),
        mesh=_sc_mesh(),
        compiler_params=pltpu.CompilerParams(needs_layout_passes=False),
        scratch_types=[pltpu.VMEM((2 * _CCH,), jnp.int32),
                       pltpu.VMEM((2 * _CCH,), jnp.int32),
                       pltpu.VMEM((2 * _CCH,), jnp.int32),
                       pltpu.VMEM((2 * _CCH, D), jnp.float32),
                       pltpu.VMEM((_CCH, D), jnp.float32),
                       pltpu.SemaphoreType.DMA,
                       pltpu.SemaphoreType.DMA],
    )


def kernel(hidden_states, self_attn_weights, gate_w, w1, w2, w3):
    x2d = hidden_states.reshape(T, D)

    rs_col = _attn_row_sums(self_attn_weights)
    logits, rw0, rw1, s0, s1, l1 = _router(x2d, gate_w)
    w0c, w1c = _rank_weights(rs_col.reshape(1, S), rs_col,
                             l1.reshape(1, T), l1, rw0, rw1)

    s0f = s0.reshape(T)
    s1f = s1.reshape(T)
    w0f = w0c.reshape(T)
    w1f = w1c.reshape(T)

    disp, pstage, counts16, wsorted = _dispatch_kernel()(s0f, s1f, w0f, w1f)
    xs = _gather_x_kernel()(x2d, disp, counts16)

    actk, blkk, wselk = _blockmap(counts16)
    act = actk.reshape(128)
    blkid = blkk.reshape(128)
    wsel = wselk.reshape(128)

    y = _grouped_mlp(act, blkid, wsel, xs, wsorted.reshape(-1, 1), w1, w3, w2)
    p2 = _reduce_pstage(pstage)
    final = _combine_kernel()(y, p2)
    return final, logits


# speedup vs baseline: 1.0073x; 1.0073x over previous
import functools

import jax
import jax.numpy as jnp
from jax import lax
from jax.experimental import pallas as pl
from jax.experimental.pallas import tpu as pltpu
from jax.experimental.pallas import tpu_sc as plsc

E = 8
TOP_K = 2
D = 1024
FF = 2048
S = 2048
T = 2048
H = 12
BETA = 0.5
KL = int(0.1 * T)
KH = int(0.1 * T)

C = T
BT = 256
NBPE = C // BT
MAXB = E * NBPE
BS_A = 1024


def _attn_sum_body(a_ref, o_ref):
    h = pl.program_id(1)
    s = jnp.sum(a_ref[0, 0], axis=1, keepdims=True)
    o_ref[...] = jnp.where(h == 0, s, o_ref[...] + s)


def _attn_row_sums(attn):
    return pl.pallas_call(
        _attn_sum_body,
        grid=(S // BS_A, H),
        in_specs=[pl.BlockSpec((1, 1, BS_A, S), lambda i, h: (0, h, i, 0))],
        out_specs=pl.BlockSpec((BS_A, 1), lambda i, h: (i, 0)),
        out_shape=jax.ShapeDtypeStruct((S, 1), jnp.float32),
    )(attn)


def _router_body(x_ref, g_ref, logits_ref, rw0_ref, rw1_ref, s0_ref, s1_ref,
                 l1_ref):
    x = x_ref[...]
    g = g_ref[...]
    logits = lax.dot_general(x, g, (((1,), (1,)), ((), ())),
                             preferred_element_type=jnp.float32)
    logits_ref[...] = logits
    m = jnp.max(logits, axis=1, keepdims=True)
    ex = jnp.exp(logits - m)
    probs = ex / jnp.sum(ex, axis=1, keepdims=True)
    cols = lax.broadcasted_iota(jnp.int32, probs.shape, 1)
    p0 = jnp.max(probs, axis=1, keepdims=True)
    i0 = jnp.min(jnp.where(probs == p0, cols, E), axis=1, keepdims=True)
    probs2 = jnp.where(cols == i0, -1.0, probs)
    p1 = jnp.max(probs2, axis=1, keepdims=True)
    i1 = jnp.min(jnp.where(probs2 == p1, cols, E), axis=1, keepdims=True)
    rw0_ref[...] = p0
    rw1_ref[...] = p1
    s0_ref[...] = i0
    s1_ref[...] = i1
    l1_ref[...] = jnp.sum(jnp.abs(x), axis=1, keepdims=True)


def _router(x2d, gate_w):
    f1 = jax.ShapeDtypeStruct((T, 1), jnp.float32)
    i1 = jax.ShapeDtypeStruct((T, 1), jnp.int32)
    col = pl.BlockSpec((BT, 1), lambda i: (i, 0))
    return pl.pallas_call(
        _router_body,
        grid=(T // BT,),
        in_specs=[pl.BlockSpec((BT, D), lambda i: (i, 0)),
                  pl.BlockSpec((E, D), lambda i: (0, 0))],
        out_specs=[pl.BlockSpec((BT, E), lambda i: (i, 0)),
                   col, col, col, col, col],
        out_shape=[jax.ShapeDtypeStruct((T, E), jnp.float32),
                   f1, f1, i1, i1, f1],
    )(x2d, gate_w)


def _rank_body(rs_row_ref, rs_col_ref, l1_row_ref, l1_col_ref, rw0_ref,
               rw1_ref, w0_ref, w1_ref):
    i = pl.program_id(0)
    j_full = lax.broadcasted_iota(jnp.int32, (1, T), 1)
    i_col = lax.broadcasted_iota(jnp.int32, (BT, 1), 0) + i * BT
    real_row = (S - j_full).astype(jnp.float32)
    real_col = (S - i_col).astype(jnp.float32)
    s_full = rs_row_ref[...] / 12.0 / real_row * l1_row_ref[...]
    s_col = rs_col_ref[...] / 12.0 / real_col * l1_col_ref[...]
    gt = jnp.sum((s_full > s_col).astype(jnp.int32), axis=1, keepdims=True)
    lt = jnp.sum((s_full < s_col).astype(jnp.int32), axis=1, keepdims=True)
    eqb = jnp.sum(((s_full == s_col) & (j_full < i_col)).astype(jnp.int32),
                  axis=1, keepdims=True)
    keep = (gt + eqb) < KH
    drop = (lt + eqb) < KL
    rw0 = rw0_ref[...]
    rw1 = rw1_ref[...]
    new0 = jnp.ones_like(rw0)
    new1 = jnp.where(rw1 < BETA * rw0, 0.0, 1.0)
    new0 = jnp.where(drop, 0.0, new0)
    new1 = jnp.where(drop, 0.0, new1)
    new0 = jnp.where(keep, 1.0, new0)
    new1 = jnp.where(keep, 1.0, new1)
    rm0 = rw0 * new0
    rm1 = rw1 * new1
    denom = rm0 + rm1
    safe = jnp.where(denom > 0, denom, 1.0)
    w0_ref[...] = jnp.where(denom > 0, rm0 / safe, 0.0)
    w1_ref[...] = jnp.where(denom > 0, rm1 / safe, 0.0)


def _rank_weights(rs_row, rs_col, l1_row, l1_col, rw0, rw1):
    f1 = jax.ShapeDtypeStruct((T, 1), jnp.float32)
    col = pl.BlockSpec((BT, 1), lambda i: (i, 0))
    row = pl.BlockSpec((1, T), lambda i: (0, 0))
    return pl.pallas_call(
        _rank_body,
        grid=(T // BT,),
        in_specs=[row, col, row, col, col, col],
        out_specs=[col, col],
        out_shape=[f1, f1],
    )(rs_row, rs_col, l1_row, l1_col, rw0, rw1)


@functools.lru_cache(maxsize=None)
def _sc_mesh():
    return plsc.VectorSubcoreMesh(core_axis_name="c", subcore_axis_name="s")


def _dispatch_body(s0_h, s1_h, w0_h, w1_h, disp_h, pst_h, cnt_h, ws_h,
                   s0_v, s1_v, w0_v, w1_v, seg_v, p0_v, p1_v, cv_v, wsor_v):
    wid = lax.axis_index("c") * 16 + lax.axis_index("s")

    @pl.when(wid < E)
    def _():
        e = wid
        pltpu.sync_copy(s0_h, s0_v)
        pltpu.sync_copy(s1_h, s1_v)
        pltpu.sync_copy(w0_h, w0_v)
        pltpu.sync_copy(w1_h, w1_v)
        z = jnp.zeros((16,), jnp.int32)
        zf = jnp.zeros((16,), jnp.float32)

        def zbody(i, carry):
            seg_v[pl.ds(i * 16, 16)] = z
            p0_v[pl.ds(i * 16, 16)] = z
            p1_v[pl.ds(i * 16, 16)] = z
            wsor_v[pl.ds(i * 16, 16)] = zf
            return carry

        lax.fori_loop(0, T // 16, zbody, 0)

        def body(ci, cnt):
            lanes = ci * 16 + lax.iota(jnp.int32, 16)
            s0 = s0_v[pl.ds(ci * 16, 16)]
            w0 = w0_v[pl.ds(ci * 16, 16)]
            m0 = (s0 == e) & (w0 > 0.0)
            pos0 = cnt + plsc.cumsum(m0.astype(jnp.int32)) - 1
            plsc.store_scatter(seg_v, [pos0], lanes, mask=m0)
            plsc.store_scatter(wsor_v, [pos0], w0, mask=m0)
            plsc.store_scatter(p0_v, [lanes], pos0 + (e * C + 1), mask=m0)
            cnt = cnt + jnp.sum(m0.astype(jnp.int32))
            s1 = s1_v[pl.ds(ci * 16, 16)]
            w1 = w1_v[pl.ds(ci * 16, 16)]
            m1 = (s1 == e) & (w1 > 0.0)
            pos1 = cnt + plsc.cumsum(m1.astype(jnp.int32)) - 1
            plsc.store_scatter(seg_v, [pos1], lanes, mask=m1)
            plsc.store_scatter(wsor_v, [pos1], w1, mask=m1)
            plsc.store_scatter(p1_v, [lanes], pos1 + (e * C + 1), mask=m1)
            cnt = cnt + jnp.sum(m1.astype(jnp.int32))
            return cnt

        cnt = lax.fori_loop(0, T // 16, body, 0)
        cv_v[...] = jnp.full((16,), cnt, jnp.int32)
        pltpu.sync_copy(cv_v, cnt_h.at[e])
        pltpu.sync_copy(seg_v, disp_h.at[pl.ds(e * C, C)])
        pltpu.sync_copy(wsor_v, ws_h.at[pl.ds(e * C, C)])
        pltpu.sync_copy(p0_v, pst_h.at[e, 0])
        pltpu.sync_copy(p1_v, pst_h.at[e, 1])


@functools.lru_cache(maxsize=None)
def _dispatch_kernel():
    return pl.kernel(
        _dispatch_body,
        out_type=(jax.ShapeDtypeStruct((E * C,), jnp.int32),
                  jax.ShapeDtypeStruct((E, 2, T), jnp.int32),
                  jax.ShapeDtypeStruct((E, 16), jnp.int32),
                  jax.ShapeDtypeStruct((E * C + BT,), jnp.float32)),
        mesh=_sc_mesh(),
        compiler_params=pltpu.CompilerParams(needs_layout_passes=False),
        scratch_types=[pltpu.VMEM((T,), jnp.int32),
                       pltpu.VMEM((T,), jnp.int32),
                       pltpu.VMEM((T,), jnp.float32),
                       pltpu.VMEM((T,), jnp.float32),
                       pltpu.VMEM((C,), jnp.int32),
                       pltpu.VMEM((T,), jnp.int32),
                       pltpu.VMEM((T,), jnp.int32),
                       pltpu.VMEM((16,), jnp.int32),
                       pltpu.VMEM((C,), jnp.float32)],
    )


_GCH = 64


def _gather_body(x_h, disp_h, cnt_h, xs_h, idx_v, rows_v, cv_v, sem):
    wid = lax.axis_index("c") * 16 + lax.axis_index("s")
    e = wid // 4
    q = wid % 4
    pltpu.sync_copy(cnt_h.at[e], cv_v)
    cnt = cv_v[...][0]
    nslots = ((cnt + BT - 1) // BT) * BT

    def body(jj, carry):
        start = (jj * 4 + q) * _GCH

        @pl.when(start < nslots)
        def _():
            pltpu.sync_copy(disp_h.at[pl.ds(e * C + start, _GCH)], idx_v)
            pltpu.async_copy(x_h.at[idx_v], rows_v, sem).wait()
            pltpu.sync_copy(rows_v, xs_h.at[pl.ds(e * C + start, _GCH)])
        return carry

    lax.fori_loop(0, (C // 4) // _GCH, body, 0)


@functools.lru_cache(maxsize=None)
def _gather_x_kernel():
    return pl.kernel(
        _gather_body,
        out_type=jax.ShapeDtypeStruct((E * C + BT, D), jnp.float32),
        mesh=_sc_mesh(),
        compiler_params=pltpu.CompilerParams(needs_layout_passes=False),
        scratch_types=[pltpu.VMEM((_GCH,), jnp.int32),
                       pltpu.VMEM((_GCH, D), jnp.float32),
                       pltpu.VMEM((16,), jnp.int32),
                       pltpu.SemaphoreType.DMA],
    )


def _mlp_body(act_s, blk_s, wsel_s, x_ref, ws_ref, w1_ref, w3_ref, w2_ref,
              y_ref):
    g = pl.program_id(0)

    @pl.when(act_s[g] == 1)
    def _():
        x = x_ref[...]
        h1 = lax.dot_general(x, w1_ref[0], (((1,), (1,)), ((), ())),
                             preferred_element_type=jnp.float32)
        h3 = lax.dot_general(x, w3_ref[0], (((1,), (1,)), ((), ())),
                             preferred_element_type=jnp.float32)
        a = (h1 * jax.nn.sigmoid(h1)) * h3
        y = lax.dot_general(a, w2_ref[0], (((1,), (1,)), ((), ())),
                            preferred_element_type=jnp.float32)
        y_ref[...] = y * ws_ref[...]

    @pl.when(act_s[g] == 2)
    def _():
        y_ref[...] = jnp.zeros((BT, D), jnp.float32)


def _grouped_mlp(act, blkid, wsel, xs, ws2d, w1, w3, w2):
    grid_spec = pltpu.PrefetchScalarGridSpec(
        num_scalar_prefetch=3,
        grid=(MAXB + 1,),
        in_specs=[
            pl.BlockSpec((BT, D), lambda g, a, b, w: (b[g], 0)),
            pl.BlockSpec((BT, 1), lambda g, a, b, w: (b[g], 0)),
            pl.BlockSpec((1, FF, D), lambda g, a, b, w: (w[g], 0, 0)),
            pl.BlockSpec((1, FF, D), lambda g, a, b, w: (w[g], 0, 0)),
            pl.BlockSpec((1, D, FF), lambda g, a, b, w: (w[g], 0, 0)),
        ],
        out_specs=pl.BlockSpec((BT, D), lambda g, a, b, w: (b[g], 0)),
    )
    return pl.pallas_call(
        _mlp_body,
        grid_spec=grid_spec,
        out_shape=jax.ShapeDtypeStruct(((MAXB + 1) * BT, D), jnp.float32),
        compiler_params=pltpu.CompilerParams(vmem_limit_bytes=100 * 1024 * 1024),
    )(act, blkid, wsel, xs, ws2d, w1, w3, w2)


def _blockmap_body(cnt_ref, act_ref, blk_ref, wsel_ref):
    c = cnt_ref[...][:, 0:1]
    nb = (c + BT - 1) // BT
    er = lax.broadcasted_iota(jnp.int32, (E, E), 0)
    ec = lax.broadcasted_iota(jnp.int32, (E, E), 1)
    nbr = jnp.broadcast_to(nb.reshape(1, E), (E, E))
    cum_incl = jnp.sum(jnp.where(ec <= er, nbr, 0), axis=1, keepdims=True)
    g = lax.broadcasted_iota(jnp.int32, (1, 128), 1)
    ge = g >= cum_incl
    eg = jnp.sum(ge.astype(jnp.int32), axis=0, keepdims=True)
    cume = jnp.sum(jnp.where(ge, jnp.broadcast_to(nb, (E, 128)), 0),
                   axis=0, keepdims=True)
    tot = jnp.sum(nb)
    egc = jnp.minimum(eg, E - 1)
    act = jnp.where(g < tot, 1, 0)
    act = jnp.where(g == MAXB, 2, act)
    act_ref[...] = act
    blk_ref[...] = jnp.where(act == 1, egc * NBPE + (g - cume), MAXB)
    wsel_ref[...] = jnp.where(act == 1, egc, 0)


def _blockmap(counts16):
    o = jax.ShapeDtypeStruct((1, 128), jnp.int32)
    full = pl.BlockSpec((1, 128), lambda: (0, 0))
    return pl.pallas_call(
        _blockmap_body,
        in_specs=[pl.BlockSpec((E, 16), lambda: (0, 0))],
        out_specs=[full, full, full],
        out_shape=[o, o, o],
    )(counts16)


def _pred_body(pst_ref, p2_ref):
    s = jnp.sum(pst_ref[...], axis=0) - 1
    p2_ref[...] = jnp.where(s < 0, E * C, s)


def _reduce_pstage(pstage):
    return pl.pallas_call(
        _pred_body,
        in_specs=[pl.BlockSpec((E, 2, T), lambda: (0, 0, 0))],
        out_specs=pl.BlockSpec((2, T), lambda: (0, 0)),
        out_shape=jax.ShapeDtypeStruct((2, T), jnp.int32),
    )(pstage)


_QT = 16


def _combine_body(y_h, p2_h, out_h,
                  p0_v, p1_v, idx0_v, idx1_v, rowsa_v, rowsb_v, acc_v,
                  sem0, sem1):
    wid = lax.axis_index("c") * 16 + lax.axis_index("s")
    t0 = wid * 64
    pltpu.sync_copy(p2_h.at[0, pl.ds(t0, 64)], p0_v)
    pltpu.sync_copy(p2_h.at[1, pl.ds(t0, 64)], p1_v)

    def build(qt, idx_v):
        lanes = qt * _QT + lax.iota(jnp.int32, 16)
        dpos = 2 * lax.iota(jnp.int32, 16)
        plsc.store_scatter(idx_v, [dpos], plsc.load_gather(p0_v, [lanes]))
        plsc.store_scatter(idx_v, [dpos + 1], plsc.load_gather(p1_v, [lanes]))

    def finish(qt, rows_v):
        def accum(r, carry):
            for i in range(D // 16):
                sl = pl.ds(i * 16, 16)
                acc_v[r, sl] = rows_v[2 * r, sl] + rows_v[2 * r + 1, sl]
            return carry

        lax.fori_loop(0, _QT, accum, 0)
        pltpu.sync_copy(acc_v, out_h.at[0, pl.ds(t0 + qt * _QT, _QT)])

    build(0, idx0_v)
    h0 = pltpu.async_copy(y_h.at[idx0_v], rowsa_v, sem0)
    build(1, idx1_v)
    h1 = pltpu.async_copy(y_h.at[idx1_v], rowsb_v, sem1)
    h0.wait()
    finish(0, rowsa_v)
    build(2, idx0_v)
    h2 = pltpu.async_copy(y_h.at[idx0_v], rowsa_v, sem0)
    h1.wait()
    finish(1, rowsb_v)
    build(3, idx1_v)
    h3 = pltpu.async_copy(y_h.at[idx1_v], rowsb_v, sem1)
    h2.wait()
    finish(2, rowsa_v)
    h3.wait()
    finish(3, rowsb_v)


@functools.lru_cache(maxsize=None)
def _combine_kernel():
    return pl.kernel(
        _combine_body,
        out_type=jax.ShapeDtypeStruct((1, S, D), jnp.float32),
        mesh=_sc_mesh(),
        compiler_params=pltpu.CompilerParams(needs_layout_passes=False),
        scratch_types=[pltpu.VMEM((4 * _QT,), jnp.int32),
                       pltpu.VMEM((4 * _QT,), jnp.int32),
                       pltpu.VMEM((2 * _QT,), jnp.int32),
                       pltpu.VMEM((2 * _QT,), jnp.int32),
                       pltpu.VMEM((2 * _QT, D), jnp.float32),
                       pltpu.VMEM((2 * _QT, D), jnp.float32),
                       pltpu.VMEM((_QT, D), jnp.float32),
                       pltpu.SemaphoreType.DMA,
                       pltpu.SemaphoreType.DMA],
    )


def kernel(hidden_states, self_attn_weights, gate_w, w1, w2, w3):
    x2d = hidden_states.reshape(T, D)

    rs_col = _attn_row_sums(self_attn_weights)
    logits, rw0, rw1, s0, s1, l1 = _router(x2d, gate_w)
    w0c, w1c = _rank_weights(rs_col.reshape(1, S), rs_col,
                             l1.reshape(1, T), l1, rw0, rw1)

    s0f = s0.reshape(T)
    s1f = s1.reshape(T)
    w0f = w0c.reshape(T)
    w1f = w1c.reshape(T)

    disp, pstage, counts16, wsorted = _dispatch_kernel()(s0f, s1f, w0f, w1f)
    xs = _gather_x_kernel()(x2d, disp, counts16)

    actk, blkk, wselk = _blockmap(counts16)
    act = actk.reshape(128)
    blkid = blkk.reshape(128)
    wsel = wselk.reshape(128)

    y = _grouped_mlp(act, blkid, wsel, xs, wsorted.reshape(-1, 1), w1, w3, w2)
    p2 = _reduce_pstage(pstage)
    final = _combine_kernel()(y, p2)
    return final, logits


# speedup vs baseline: 1.0115x; 1.0042x over previous
import functools

import jax
import jax.numpy as jnp
from jax import lax
from jax.experimental import pallas as pl
from jax.experimental.pallas import tpu as pltpu
from jax.experimental.pallas import tpu_sc as plsc

E = 8
TOP_K = 2
D = 1024
FF = 2048
S = 2048
T = 2048
H = 12
BETA = 0.5
KL = int(0.1 * T)
KH = int(0.1 * T)

C = T
BT = 256
NBPE = C // BT
MAXB = E * NBPE
BS_A = 1024


def _attn_sum_body(a_ref, o_ref):
    h = pl.program_id(1)
    s = jnp.sum(a_ref[0, 0], axis=1, keepdims=True)
    o_ref[...] = jnp.where(h == 0, s, o_ref[...] + s)


def _attn_row_sums(attn):
    return pl.pallas_call(
        _attn_sum_body,
        grid=(S // BS_A, H),
        in_specs=[pl.BlockSpec((1, 1, BS_A, S), lambda i, h: (0, h, i, 0))],
        out_specs=pl.BlockSpec((BS_A, 1), lambda i, h: (i, 0)),
        out_shape=jax.ShapeDtypeStruct((S, 1), jnp.float32),
    )(attn)


def _router_body(x_ref, g_ref, logits_ref, rw0_ref, rw1_ref, s0_ref, s1_ref,
                 l1_ref):
    x = x_ref[0]
    g = g_ref[...]
    logits = lax.dot_general(x, g, (((1,), (1,)), ((), ())),
                             preferred_element_type=jnp.float32)
    logits_ref[...] = logits
    m = jnp.max(logits, axis=1, keepdims=True)
    ex = jnp.exp(logits - m)
    probs = ex / jnp.sum(ex, axis=1, keepdims=True)
    cols = lax.broadcasted_iota(jnp.int32, probs.shape, 1)
    p0 = jnp.max(probs, axis=1, keepdims=True)
    i0 = jnp.min(jnp.where(probs == p0, cols, E), axis=1, keepdims=True)
    probs2 = jnp.where(cols == i0, -1.0, probs)
    p1 = jnp.max(probs2, axis=1, keepdims=True)
    i1 = jnp.min(jnp.where(probs2 == p1, cols, E), axis=1, keepdims=True)
    rw0_ref[...] = p0
    rw1_ref[...] = p1
    s0_ref[...] = i0
    s1_ref[...] = i1
    l1_ref[...] = jnp.sum(jnp.abs(x), axis=1, keepdims=True)


def _router(x2d, gate_w):
    f1 = jax.ShapeDtypeStruct((T, 1), jnp.float32)
    i1 = jax.ShapeDtypeStruct((T, 1), jnp.int32)
    col = pl.BlockSpec((BT, 1), lambda i: (i, 0))
    return pl.pallas_call(
        _router_body,
        grid=(T // BT,),
        in_specs=[pl.BlockSpec((1, BT, D), lambda i: (0, i, 0)),
                  pl.BlockSpec((E, D), lambda i: (0, 0))],
        out_specs=[pl.BlockSpec((BT, E), lambda i: (i, 0)),
                   col, col, col, col, col],
        out_shape=[jax.ShapeDtypeStruct((T, E), jnp.float32),
                   f1, f1, i1, i1, f1],
    )(x2d, gate_w)


def _rank_body(rs_row_ref, rs_col_ref, l1_row_ref, l1_col_ref, rw0_ref,
               rw1_ref, w0_ref, w1_ref):
    i = pl.program_id(0)
    j_full = lax.broadcasted_iota(jnp.int32, (1, T), 1)
    i_col = lax.broadcasted_iota(jnp.int32, (BT, 1), 0) + i * BT
    real_row = (S - j_full).astype(jnp.float32)
    real_col = (S - i_col).astype(jnp.float32)
    s_full = rs_row_ref[...] / 12.0 / real_row * l1_row_ref[...]
    s_col = rs_col_ref[...] / 12.0 / real_col * l1_col_ref[...]
    gt = jnp.sum((s_full > s_col).astype(jnp.int32), axis=1, keepdims=True)
    lt = jnp.sum((s_full < s_col).astype(jnp.int32), axis=1, keepdims=True)
    eqb = jnp.sum(((s_full == s_col) & (j_full < i_col)).astype(jnp.int32),
                  axis=1, keepdims=True)
    keep = (gt + eqb) < KH
    drop = (lt + eqb) < KL
    rw0 = rw0_ref[...]
    rw1 = rw1_ref[...]
    new0 = jnp.ones_like(rw0)
    new1 = jnp.where(rw1 < BETA * rw0, 0.0, 1.0)
    new0 = jnp.where(drop, 0.0, new0)
    new1 = jnp.where(drop, 0.0, new1)
    new0 = jnp.where(keep, 1.0, new0)
    new1 = jnp.where(keep, 1.0, new1)
    rm0 = rw0 * new0
    rm1 = rw1 * new1
    denom = rm0 + rm1
    safe = jnp.where(denom > 0, denom, 1.0)
    w0_ref[...] = jnp.where(denom > 0, rm0 / safe, 0.0)
    w1_ref[...] = jnp.where(denom > 0, rm1 / safe, 0.0)


def _rank_weights(rs_row, rs_col, l1_row, l1_col, rw0, rw1):
    f1 = jax.ShapeDtypeStruct((T, 1), jnp.float32)
    col = pl.BlockSpec((BT, 1), lambda i: (i, 0))
    row = pl.BlockSpec((1, T), lambda i: (0, 0))
    return pl.pallas_call(
        _rank_body,
        grid=(T // BT,),
        in_specs=[row, col, row, col, col, col],
        out_specs=[col, col],
        out_shape=[f1, f1],
    )(rs_row, rs_col, l1_row, l1_col, rw0, rw1)


@functools.lru_cache(maxsize=None)
def _sc_mesh():
    return plsc.VectorSubcoreMesh(core_axis_name="c", subcore_axis_name="s")


def _dispatch_body(s0_h, s1_h, w0_h, w1_h, disp_h, pst_h, cnt_h, ws_h,
                   s0_v, s1_v, w0_v, w1_v, seg_v, p0_v, p1_v, cv_v, wsor_v):
    wid = lax.axis_index("c") * 16 + lax.axis_index("s")

    @pl.when(wid < E)
    def _():
        e = wid
        pltpu.sync_copy(s0_h, s0_v)
        pltpu.sync_copy(s1_h, s1_v)
        pltpu.sync_copy(w0_h, w0_v)
        pltpu.sync_copy(w1_h, w1_v)
        z = jnp.zeros((16,), jnp.int32)
        zf = jnp.zeros((16,), jnp.float32)

        def zbody(i, carry):
            seg_v[pl.ds(i * 16, 16)] = z
            p0_v[pl.ds(i * 16, 16)] = z
            p1_v[pl.ds(i * 16, 16)] = z
            wsor_v[pl.ds(i * 16, 16)] = zf
            return carry

        lax.fori_loop(0, T // 16, zbody, 0)

        def body(ci, cnt):
            lanes = ci * 16 + lax.iota(jnp.int32, 16)
            s0 = s0_v[pl.ds(ci * 16, 16)]
            w0 = w0_v[pl.ds(ci * 16, 16)]
            m0 = (s0 == e) & (w0 > 0.0)
            pos0 = cnt + plsc.cumsum(m0.astype(jnp.int32)) - 1
            plsc.store_scatter(seg_v, [pos0], lanes, mask=m0)
            plsc.store_scatter(wsor_v, [pos0], w0, mask=m0)
            plsc.store_scatter(p0_v, [lanes], pos0 + (e * C + 1), mask=m0)
            cnt = cnt + jnp.sum(m0.astype(jnp.int32))
            s1 = s1_v[pl.ds(ci * 16, 16)]
            w1 = w1_v[pl.ds(ci * 16, 16)]
            m1 = (s1 == e) & (w1 > 0.0)
            pos1 = cnt + plsc.cumsum(m1.astype(jnp.int32)) - 1
            plsc.store_scatter(seg_v, [pos1], lanes, mask=m1)
            plsc.store_scatter(wsor_v, [pos1], w1, mask=m1)
            plsc.store_scatter(p1_v, [lanes], pos1 + (e * C + 1), mask=m1)
            cnt = cnt + jnp.sum(m1.astype(jnp.int32))
            return cnt

        cnt = lax.fori_loop(0, T // 16, body, 0)
        cv_v[...] = jnp.full((16,), cnt, jnp.int32)
        pltpu.sync_copy(cv_v, cnt_h.at[e])
        pltpu.sync_copy(seg_v, disp_h.at[pl.ds(e * C, C)])
        pltpu.sync_copy(wsor_v, ws_h.at[pl.ds(e * C, C)])
        pltpu.sync_copy(p0_v, pst_h.at[e, 0])
        pltpu.sync_copy(p1_v, pst_h.at[e, 1])


@functools.lru_cache(maxsize=None)
def _dispatch_kernel():
    return pl.kernel(
        _dispatch_body,
        out_type=(jax.ShapeDtypeStruct((E * C,), jnp.int32),
                  jax.ShapeDtypeStruct((E, 2, T), jnp.int32),
                  jax.ShapeDtypeStruct((E, 16), jnp.int32),
                  jax.ShapeDtypeStruct((E * C + BT,), jnp.float32)),
        mesh=_sc_mesh(),
        compiler_params=pltpu.CompilerParams(needs_layout_passes=False),
        scratch_types=[pltpu.VMEM((T,), jnp.int32),
                       pltpu.VMEM((T,), jnp.int32),
                       pltpu.VMEM((T,), jnp.float32),
                       pltpu.VMEM((T,), jnp.float32),
                       pltpu.VMEM((C,), jnp.int32),
                       pltpu.VMEM((T,), jnp.int32),
                       pltpu.VMEM((T,), jnp.int32),
                       pltpu.VMEM((16,), jnp.int32),
                       pltpu.VMEM((C,), jnp.float32)],
    )


_GCH = 64


def _gather_body(x_h, disp_h, cnt_h, xs_h, idx_v, rows_v, cv_v, sem):
    wid = lax.axis_index("c") * 16 + lax.axis_index("s")
    e = wid // 4
    q = wid % 4
    pltpu.sync_copy(cnt_h.at[e], cv_v)
    cnt = cv_v[...][0]
    nslots = ((cnt + BT - 1) // BT) * BT

    def body(jj, carry):
        start = (jj * 4 + q) * _GCH

        @pl.when(start < nslots)
        def _():
            pltpu.sync_copy(disp_h.at[pl.ds(e * C + start, _GCH)], idx_v)
            pltpu.async_copy(x_h.at[0].at[idx_v], rows_v, sem).wait()
            pltpu.sync_copy(rows_v, xs_h.at[pl.ds(e * C + start, _GCH)])
        return carry

    lax.fori_loop(0, (C // 4) // _GCH, body, 0)


@functools.lru_cache(maxsize=None)
def _gather_x_kernel():
    return pl.kernel(
        _gather_body,
        out_type=jax.ShapeDtypeStruct((E * C + BT, D), jnp.float32),
        mesh=_sc_mesh(),
        compiler_params=pltpu.CompilerParams(needs_layout_passes=False),
        scratch_types=[pltpu.VMEM((_GCH,), jnp.int32),
                       pltpu.VMEM((_GCH, D), jnp.float32),
                       pltpu.VMEM((16,), jnp.int32),
                       pltpu.SemaphoreType.DMA],
    )


def _mlp_body(act_s, blk_s, wsel_s, x_ref, ws_ref, w1_ref, w3_ref, w2_ref,
              y_ref):
    g = pl.program_id(0)

    @pl.when(act_s[g] == 1)
    def _():
        x = x_ref[...]
        h1 = lax.dot_general(x, w1_ref[0], (((1,), (1,)), ((), ())),
                             preferred_element_type=jnp.float32)
        h3 = lax.dot_general(x, w3_ref[0], (((1,), (1,)), ((), ())),
                             preferred_element_type=jnp.float32)
        a = (h1 * jax.nn.sigmoid(h1)) * h3
        y = lax.dot_general(a, w2_ref[0], (((1,), (1,)), ((), ())),
                            preferred_element_type=jnp.float32)
        y_ref[...] = y * ws_ref[...]

    @pl.when(act_s[g] == 2)
    def _():
        y_ref[...] = jnp.zeros((BT, D), jnp.float32)


def _grouped_mlp(act, blkid, wsel, xs, ws2d, w1, w3, w2):
    grid_spec = pltpu.PrefetchScalarGridSpec(
        num_scalar_prefetch=3,
        grid=(MAXB + 1,),
        in_specs=[
            pl.BlockSpec((BT, D), lambda g, a, b, w: (b[g], 0)),
            pl.BlockSpec((BT, 1), lambda g, a, b, w: (b[g], 0)),
            pl.BlockSpec((1, FF, D), lambda g, a, b, w: (w[g], 0, 0)),
            pl.BlockSpec((1, FF, D), lambda g, a, b, w: (w[g], 0, 0)),
            pl.BlockSpec((1, D, FF), lambda g, a, b, w: (w[g], 0, 0)),
        ],
        out_specs=pl.BlockSpec((BT, D), lambda g, a, b, w: (b[g], 0)),
    )
    return pl.pallas_call(
        _mlp_body,
        grid_spec=grid_spec,
        out_shape=jax.ShapeDtypeStruct(((MAXB + 1) * BT, D), jnp.float32),
        compiler_params=pltpu.CompilerParams(vmem_limit_bytes=100 * 1024 * 1024),
    )(act, blkid, wsel, xs, ws2d, w1, w3, w2)


def _blockmap_body(cnt_ref, act_ref, blk_ref, wsel_ref):
    c = cnt_ref[...][:, 0:1]
    nb = (c + BT - 1) // BT
    er = lax.broadcasted_iota(jnp.int32, (E, E), 0)
    ec = lax.broadcasted_iota(jnp.int32, (E, E), 1)
    nbr = jnp.broadcast_to(nb.reshape(1, E), (E, E))
    cum_incl = jnp.sum(jnp.where(ec <= er, nbr, 0), axis=1, keepdims=True)
    g = lax.broadcasted_iota(jnp.int32, (1, 128), 1)
    ge = g >= cum_incl
    eg = jnp.sum(ge.astype(jnp.int32), axis=0, keepdims=True)
    cume = jnp.sum(jnp.where(ge, jnp.broadcast_to(nb, (E, 128)), 0),
                   axis=0, keepdims=True)
    tot = jnp.sum(nb)
    egc = jnp.minimum(eg, E - 1)
    act = jnp.where(g < tot, 1, 0)
    act = jnp.where(g == MAXB, 2, act)
    act_ref[...] = act
    blk_ref[...] = jnp.where(act == 1, egc * NBPE + (g - cume), MAXB)
    wsel_ref[...] = jnp.where(act == 1, egc, 0)


def _blockmap(counts16):
    o = jax.ShapeDtypeStruct((1, 128), jnp.int32)
    full = pl.BlockSpec((1, 128), lambda: (0, 0))
    return pl.pallas_call(
        _blockmap_body,
        in_specs=[pl.BlockSpec((E, 16), lambda: (0, 0))],
        out_specs=[full, full, full],
        out_shape=[o, o, o],
    )(counts16)


def _pred_body(pst_ref, p2_ref):
    s = jnp.sum(pst_ref[...], axis=0) - 1
    p2_ref[...] = jnp.where(s < 0, E * C, s)


def _reduce_pstage(pstage):
    return pl.pallas_call(
        _pred_body,
        in_specs=[pl.BlockSpec((E, 2, T), lambda: (0, 0, 0))],
        out_specs=pl.BlockSpec((2, T), lambda: (0, 0)),
        out_shape=jax.ShapeDtypeStruct((2, T), jnp.int32),
    )(pstage)


_QT = 16


def _combine_body(y_h, p2_h, out_h,
                  p0_v, p1_v, idx0_v, idx1_v, rowsa_v, rowsb_v, acc_v,
                  sem0, sem1):
    wid = lax.axis_index("c") * 16 + lax.axis_index("s")
    t0 = wid * 64
    pltpu.sync_copy(p2_h.at[0, pl.ds(t0, 64)], p0_v)
    pltpu.sync_copy(p2_h.at[1, pl.ds(t0, 64)], p1_v)

    def build(qt, idx_v):
        lanes = qt * _QT + lax.iota(jnp.int32, 16)
        dpos = 2 * lax.iota(jnp.int32, 16)
        plsc.store_scatter(idx_v, [dpos], plsc.load_gather(p0_v, [lanes]))
        plsc.store_scatter(idx_v, [dpos + 1], plsc.load_gather(p1_v, [lanes]))

    def finish(qt, rows_v):
        def accum(r, carry):
            for i in range(D // 16):
                sl = pl.ds(i * 16, 16)
                acc_v[r, sl] = rows_v[2 * r, sl] + rows_v[2 * r + 1, sl]
            return carry

        lax.fori_loop(0, _QT, accum, 0)
        pltpu.sync_copy(acc_v, out_h.at[0, pl.ds(t0 + qt * _QT, _QT)])

    build(0, idx0_v)
    h0 = pltpu.async_copy(y_h.at[idx0_v], rowsa_v, sem0)
    build(1, idx1_v)
    h1 = pltpu.async_copy(y_h.at[idx1_v], rowsb_v, sem1)
    h0.wait()
    finish(0, rowsa_v)
    build(2, idx0_v)
    h2 = pltpu.async_copy(y_h.at[idx0_v], rowsa_v, sem0)
    h1.wait()
    finish(1, rowsb_v)
    build(3, idx1_v)
    h3 = pltpu.async_copy(y_h.at[idx1_v], rowsb_v, sem1)
    h2.wait()
    finish(2, rowsa_v)
    h3.wait()
    finish(3, rowsb_v)


@functools.lru_cache(maxsize=None)
def _combine_kernel():
    return pl.kernel(
        _combine_body,
        out_type=jax.ShapeDtypeStruct((1, S, D), jnp.float32),
        mesh=_sc_mesh(),
        compiler_params=pltpu.CompilerParams(needs_layout_passes=False),
        scratch_types=[pltpu.VMEM((4 * _QT,), jnp.int32),
                       pltpu.VMEM((4 * _QT,), jnp.int32),
                       pltpu.VMEM((2 * _QT,), jnp.int32),
                       pltpu.VMEM((2 * _QT,), jnp.int32),
                       pltpu.VMEM((2 * _QT, D), jnp.float32),
                       pltpu.VMEM((2 * _QT, D), jnp.float32),
                       pltpu.VMEM((_QT, D), jnp.float32),
                       pltpu.SemaphoreType.DMA,
                       pltpu.SemaphoreType.DMA],
    )


def kernel(hidden_states, self_attn_weights, gate_w, w1, w2, w3):
    rs_col = _attn_row_sums(self_attn_weights)
    logits, rw0, rw1, s0, s1, l1 = _router(hidden_states, gate_w)
    w0c, w1c = _rank_weights(rs_col.reshape(1, S), rs_col,
                             l1.reshape(1, T), l1, rw0, rw1)

    s0f = s0.reshape(T)
    s1f = s1.reshape(T)
    w0f = w0c.reshape(T)
    w1f = w1c.reshape(T)

    disp, pstage, counts16, wsorted = _dispatch_kernel()(s0f, s1f, w0f, w1f)
    xs = _gather_x_kernel()(hidden_states, disp, counts16)

    actk, blkk, wselk = _blockmap(counts16)
    act = actk.reshape(128)
    blkid = blkk.reshape(128)
    wsel = wselk.reshape(128)

    y = _grouped_mlp(act, blkid, wsel, xs, wsorted.reshape(-1, 1), w1, w3, w2)
    p2 = _reduce_pstage(pstage)
    final = _combine_kernel()(y, p2)
    return final, logits
